# Initial kernel scaffold; baseline (speedup 1.0000x reference)
import jax
import jax.numpy as jnp
from jax.experimental import pallas as pl


def kernel(word_encs, span_idxs, W, bias):
    raise NotImplementedError("write your pallas kernel here")


# speedup vs baseline: 31.0123x; 31.0123x over previous
import jax
import jax.numpy as jnp
from jax.experimental import pallas as pl

G = 8
VCHUNK = 128


def _body(w_col_ref, st_ref, en_ref, W_ref, b_ref, o_ref):
    T = w_col_ref.shape[1]
    S = st_ref.shape[1]
    V, D = W_ref.shape
    pos = jax.lax.broadcasted_iota(jnp.int32, (S, T), 1)
    for g in range(G):
        st = st_ref[g]
        en = en_ref[g]
        span_m = ((pos >= st) & (pos < en)).astype(jnp.float32)
        wcol = w_col_ref[g]
        acc = jnp.zeros((S, D), dtype=jnp.float32)
        for vc in range(0, V, VCHUNK):
            sz = min(VCHUNK, V - vc)
            vids = vc + jax.lax.broadcasted_iota(jnp.int32, (T, sz), 1)
            oh = (wcol == vids).astype(jnp.float32)
            counts = jnp.dot(span_m, oh, preferred_element_type=jnp.float32)
            pres = jnp.minimum(counts, 1.0)
            acc = acc + jnp.dot(pres, W_ref[vc:vc + sz, :],
                                preferred_element_type=jnp.float32)
        o_ref[g] = acc + b_ref[...]


def kernel(word_encs, span_idxs, W, bias):
    B, T = word_encs.shape
    S = span_idxs.shape[1]
    V, D = W.shape
    w_col = word_encs.astype(jnp.int32).reshape(B, T, 1)
    st = span_idxs[:, :, 0].astype(jnp.int32).reshape(B, S, 1)
    en = span_idxs[:, :, 1].astype(jnp.int32).reshape(B, S, 1)
    bias2 = bias.astype(jnp.float32).reshape(1, D)
    out = pl.pallas_call(
        _body,
        grid=(B // G,),
        in_specs=[
            pl.BlockSpec((G, T, 1), lambda i: (i, 0, 0)),
            pl.BlockSpec((G, S, 1), lambda i: (i, 0, 0)),
            pl.BlockSpec((G, S, 1), lambda i: (i, 0, 0)),
            pl.BlockSpec((V, D), lambda i: (0, 0)),
            pl.BlockSpec((1, D), lambda i: (0, 0)),
        ],
        out_specs=pl.BlockSpec((G, S, D), lambda i: (i, 0, 0)),
        out_shape=jax.ShapeDtypeStruct((B, S, D), jnp.float32),
    )(w_col, st, en, W.astype(jnp.float32), bias2)
    return out


# speedup vs baseline: 139.7795x; 4.5072x over previous
import functools
import jax
import jax.numpy as jnp
from jax import lax
from jax.experimental import pallas as pl
from jax.experimental.pallas import tpu as pltpu, tpu_sc as plsc

G = 16


def _sc_gather_t(idx2, Wr, D):
    NW, rpw = idx2.shape
    L = 16
    ngrp = rpw // L
    info = plsc.get_sparse_core_info()
    nc = info.num_cores
    mesh = plsc.VectorSubcoreMesh(core_axis_name="c", subcore_axis_name="s")

    NQ = 2
    qw = rpw // NQ

    @functools.partial(
        pl.kernel,
        mesh=mesh,
        out_type=jax.ShapeDtypeStruct((NW, NQ, D, qw), jnp.float32),
        scratch_types=[
            pltpu.VMEM((rpw,), jnp.int32),
            pltpu.VMEM(Wr.shape, jnp.float32),
            pltpu.VMEM((D, rpw), jnp.float32),
        ],
        compiler_params=pltpu.CompilerParams(needs_layout_passes=False),
    )
    def k(idx_hbm, table_hbm, out_hbm, idx_v, tab_v, rows_v):
        wid = lax.axis_index("s") * nc + lax.axis_index("c")
        pltpu.sync_copy(idx_hbm.at[wid], idx_v)
        pltpu.sync_copy(table_hbm, tab_v)

        def grp(g, _):
            off = pl.multiple_of(g * L, L)
            tok = idx_v[pl.ds(off, L)]
            base = tok * D
            for c in range(D):
                vals = plsc.load_gather(tab_v, [base + c])
                rows_v[c, pl.ds(off, L)] = vals
            return 0

        lax.fori_loop(0, ngrp, grp, 0)
        for q in range(NQ):
            pltpu.sync_copy(rows_v.at[:, pl.ds(q * qw, qw)],
                            out_hbm.at[wid, q])

    return k(idx2, Wr)


def _tc_body(wc_ref, wr_ref, st_ref, en_ref, we_ref, b_ref, o_ref):
    T = wc_ref.shape[1]
    S = st_ref.shape[1]
    rr = jax.lax.broadcasted_iota(jnp.int32, (T, T), 0)
    cc = jax.lax.broadcasted_iota(jnp.int32, (T, T), 1)
    pos = jax.lax.broadcasted_iota(jnp.int32, (S, T), 1)
    for g in range(G):
        wc = wc_ref[g]
        wr = wr_ref[g]
        eq = (wc == wr) & (rr < cc)
        prev = jnp.max(jnp.where(eq, rr, -1), axis=0)
        prev2 = prev.reshape(1, T)
        st = st_ref[g]
        en = en_ref[g]
        m = (pos >= st) & (pos < en) & (prev2 < st)
        wet = we_ref[0, 0][:, g * T:(g + 1) * T]
        o_ref[g] = lax.dot_general(
            m.astype(jnp.float32), wet,
            dimension_numbers=(((1,), (1,)), ((), ())),
            preferred_element_type=jnp.float32) + b_ref[...]


def kernel(word_encs, span_idxs, W, bias):
    B, T = word_encs.shape
    S = span_idxs.shape[1]
    V, D = W.shape
    NW = 32
    w32 = word_encs.astype(jnp.int32)
    Wf = W.astype(jnp.float32)
    WEt = _sc_gather_t(w32.reshape(NW, B * T // NW),
                       Wf.reshape(V * D), D)
    w_col = w32.reshape(B, T, 1)
    w_row = w32.reshape(B, 1, T)
    st = span_idxs[:, :, 0].astype(jnp.int32).reshape(B, S, 1)
    en = span_idxs[:, :, 1].astype(jnp.int32).reshape(B, S, 1)
    bias2 = bias.astype(jnp.float32).reshape(1, D)
    bpw = B // NW
    nblk = bpw // G
    out = pl.pallas_call(
        _tc_body,
        grid=(B // G,),
        in_specs=[
            pl.BlockSpec((G, T, 1), lambda i: (i, 0, 0)),
            pl.BlockSpec((G, 1, T), lambda i: (i, 0, 0)),
            pl.BlockSpec((G, S, 1), lambda i: (i, 0, 0)),
            pl.BlockSpec((G, S, 1), lambda i: (i, 0, 0)),
            pl.BlockSpec((1, 1, D, G * T),
                         lambda i: (i // nblk, i % nblk, 0, 0)),
            pl.BlockSpec((1, D), lambda i: (0, 0)),
        ],
        out_specs=pl.BlockSpec((G, S, D), lambda i: (i, 0, 0)),
        out_shape=jax.ShapeDtypeStruct((B, S, D), jnp.float32),
    )(w_col, w_row, st, en, WEt, bias2)
    return out
